# baseline (speedup 1.0000x reference)
import jax
import jax.numpy as jnp
from jax import lax
from jax.experimental import pallas as pl
from jax.experimental.pallas import tpu as pltpu
from jax.experimental.pallas import tpu_sc as plsc

_NC = 1000
_SMOOTHING = 0.1
_CONF = 1.0 - _SMOOTHING
_EPS = _SMOOTHING / (_NC - 1)
_BLK = 2048
_N = 16384

_NWORK = 32
_SC_ROWS = 2048
_RPW = _SC_ROWS // _NWORK
_CHUNKS = _RPW // 8
_SC_STEPS = _SC_ROWS // _BLK
_CSPLIT = 992


def _sc_body(pred_hbm, out_hbm, buf_v, acc_v, sem):
    wid = lax.axis_index("s") * 2 + lax.axis_index("c")
    base = wid * _RPW

    def _chunk(ch, acc):
        row0 = pl.multiple_of(base + ch * 8, 8)
        pltpu.sync_copy(pred_hbm.at[pl.ds(row0, 8), :], buf_v)
        for r in range(8):
            for c in range(_CSPLIT // 16):
                acc = acc + buf_v[r, pl.ds(c * 16, 16)]
        return acc

    acc = lax.fori_loop(0, _CHUNKS, _chunk, jnp.zeros((16,), jnp.float32))
    acc_v[...] = acc
    pltpu.sync_copy(acc_v, out_hbm.at[wid])


def _loss_block(pred_ref, tgt_ref, out_ref):
    i = pl.program_id(0)
    ng = pl.num_programs(0)
    x = pred_ref[...]
    t = tgt_ref[...]
    m = jnp.max(x, axis=1, keepdims=True)
    s = jnp.sum(jnp.exp(x - m), axis=1, keepdims=True)
    lse = m + jnp.log(s)
    col = jax.lax.broadcasted_iota(jnp.int32, (1, _NC), 1)
    p_t = jnp.sum(jnp.where(col == t, x, 0.0), axis=1, keepdims=True)
    full_sum = jnp.sum(x)
    tail_sum = jnp.sum(jnp.where(col >= _CSPLIT, x, 0.0))
    region_sum = jnp.where(i < _SC_STEPS, tail_sum, full_sum)
    blk = (jnp.sum(lse - (_CONF - _EPS) * p_t)
           - _EPS * region_sum).reshape(1, 1)

    @pl.when(i == 0)
    def _init():
        out_ref[...] = jnp.zeros((1, 1), jnp.float32)

    out_ref[...] += blk

    @pl.when(i == ng - 1)
    def _final():
        out_ref[...] = out_ref[...] * (1.0 / _N)


def kernel(pred, target):
    n = target.shape[0]
    tgt2d = target.astype(jnp.int32).reshape(n, 1)

    mesh = plsc.VectorSubcoreMesh(core_axis_name="c", subcore_axis_name="s")
    sc_sums = pl.kernel(
        _sc_body,
        mesh=mesh,
        out_type=jax.ShapeDtypeStruct((_NWORK, 16), jnp.float32),
        scratch_types=[
            pltpu.VMEM((8, _NC), jnp.float32),
            pltpu.VMEM((16,), jnp.float32),
            pltpu.SemaphoreType.DMA,
        ],
    )(pred)

    grid = n // _BLK
    total = pl.pallas_call(
        _loss_block,
        grid=(grid,),
        in_specs=[
            pl.BlockSpec((_BLK, _NC), lambda i: (i, 0)),
            pl.BlockSpec((_BLK, 1), lambda i: (i, 0)),
        ],
        out_specs=pl.BlockSpec((1, 1), lambda i: (0, 0)),
        out_shape=jax.ShapeDtypeStruct((1, 1), jnp.float32),
    )(pred, tgt2d)

    return total[0, 0] - (_EPS / n) * jnp.sum(sc_sums)


# speedup vs baseline: 1.1911x; 1.1911x over previous
import jax
import jax.numpy as jnp
from jax.experimental import pallas as pl
from jax.experimental.pallas import tpu as pltpu

_NC = 1000
_SMOOTHING = 0.1
_CONF = 1.0 - _SMOOTHING
_EPS = _SMOOTHING / (_NC - 1)
_BLK = 2048


def _loss_block(pred_ref, tgt_ref, out_ref):
    i = pl.program_id(0)
    ng = pl.num_programs(0)
    x = pred_ref[...]
    t = tgt_ref[...]
    m = jnp.max(x, axis=1, keepdims=True)
    s = jnp.sum(jnp.exp(x - m), axis=1, keepdims=True)
    lse = m + jnp.log(s)
    sum_pred = jnp.sum(x, axis=1, keepdims=True)
    col = jax.lax.broadcasted_iota(jnp.int32, (1, _NC), 1)
    p_t = jnp.sum(jnp.where(col == t, x, 0.0), axis=1, keepdims=True)
    blk = jnp.sum(lse - _EPS * sum_pred - (_CONF - _EPS) * p_t).reshape(1, 1)

    @pl.when(i == 0)
    def _init():
        out_ref[...] = jnp.zeros((1, 1), jnp.float32)

    out_ref[...] += blk

    @pl.when(i == ng - 1)
    def _final():
        out_ref[...] = out_ref[...] * (1.0 / (_BLK * ng))


def kernel(pred, target):
    n = target.shape[0]
    tgt2d = target.astype(jnp.int32).reshape(n, 1)
    grid = n // _BLK
    total = pl.pallas_call(
        _loss_block,
        grid=(grid,),
        in_specs=[
            pl.BlockSpec((_BLK, _NC), lambda i: (i, 0)),
            pl.BlockSpec((_BLK, 1), lambda i: (i, 0)),
        ],
        out_specs=pl.BlockSpec((1, 1), lambda i: (0, 0)),
        out_shape=jax.ShapeDtypeStruct((1, 1), jnp.float32),
    )(pred, tgt2d)
    return total[0, 0]
